# baseline (device time: 1469345 ns/iter reference)
import jax
import jax.numpy as jnp
from jax import lax
from jax.experimental import pallas as pl
from jax.experimental.pallas import tpu as pltpu

N_DEV = 16


def kernel(x, w_mat):
    x = x.astype(jnp.bfloat16)
    w_mat = w_mat.astype(jnp.bfloat16)

    m_per, k = x.shape
    _, n_per = w_mat.shape
    out_m = N_DEV * m_per

    def body(x_ref, w_ref, out_ref, comm_ref, send_sems, recv_sems, credit_sem):
        my = lax.axis_index("i")
        left = lax.rem(my + N_DEV - 1, N_DEV)
        right = lax.rem(my + 1, N_DEV)

        barrier_sem = pltpu.get_barrier_semaphore()
        for nbr in (left, right):
            pl.semaphore_signal(
                barrier_sem, inc=1,
                device_id=(nbr,), device_id_type=pl.DeviceIdType.MESH,
            )
        pl.semaphore_wait(barrier_sem, 2)

        def gemm_store(origin, chunk):
            y = jnp.dot(chunk, w_ref[...], preferred_element_type=jnp.float32)
            y = y * (1.0 / (1.0 + jnp.exp(-y)))
            out_ref[pl.ds(origin * m_per, m_per), :] = y

        comm_ref[0] = x_ref[...]
        gemm_store(my, x_ref[...])

        for h in range(N_DEV - 1):
            send_slot = h % 2
            recv_slot = (h + 1) % 2
            if h >= 1:
                pl.semaphore_wait(credit_sem, 1)
            rdma = pltpu.make_async_remote_copy(
                src_ref=comm_ref.at[send_slot],
                dst_ref=comm_ref.at[recv_slot],
                send_sem=send_sems.at[send_slot],
                recv_sem=recv_sems.at[recv_slot],
                device_id=(right,),
                device_id_type=pl.DeviceIdType.MESH,
            )
            rdma.start()
            rdma.wait()
            if h < N_DEV - 2:
                pl.semaphore_signal(
                    credit_sem, inc=1,
                    device_id=(left,), device_id_type=pl.DeviceIdType.MESH,
                )
            origin = lax.rem(my + N_DEV - h - 1, N_DEV)
            gemm_store(origin, comm_ref[recv_slot])

    return pl.pallas_call(
        body,
        out_shape=jax.ShapeDtypeStruct((out_m, n_per), jnp.float32),
        in_specs=[
            pl.BlockSpec(memory_space=pltpu.VMEM),
            pl.BlockSpec(memory_space=pltpu.VMEM),
        ],
        out_specs=pl.BlockSpec(memory_space=pltpu.VMEM),
        scratch_shapes=[
            pltpu.VMEM((2, m_per, k), jnp.bfloat16),
            pltpu.SemaphoreType.DMA((2,)),
            pltpu.SemaphoreType.DMA((2,)),
            pltpu.SemaphoreType.REGULAR,
        ],
        compiler_params=pltpu.CompilerParams(collective_id=0),
    )(x, w_mat)


# device time: 745290 ns/iter; 1.9715x vs baseline; 1.9715x over previous
import jax
import jax.numpy as jnp
from jax import lax
from jax.experimental import pallas as pl
from jax.experimental.pallas import tpu as pltpu

N_DEV = 16
NSLOT = 4

RING = [0, 1, 5, 9, 13, 14, 10, 6, 2, 3, 7, 11, 15, 12, 8, 4]
INV = [0] * N_DEV
for _j, _p in enumerate(RING):
    INV[_p] = _j
NEXT = [RING[(INV[p] + 1) % N_DEV] for p in range(N_DEV)]
PREV = [RING[(INV[p] - 1) % N_DEV] for p in range(N_DEV)]


def kernel(x, w_mat):
    x = x.astype(jnp.bfloat16)
    w_mat = w_mat.astype(jnp.bfloat16)

    m_per, k = x.shape
    _, n_per = w_mat.shape
    m_half = m_per // 2
    out_m = N_DEV * m_per

    my = lax.axis_index("i")
    ring_arr = jnp.asarray(RING, jnp.int32)
    r = jnp.asarray(INV, jnp.int32)[my]
    hops = jnp.arange(N_DEV - 1, dtype=jnp.int32)
    nbrs = jnp.stack(
        [jnp.asarray(NEXT, jnp.int32)[my],
         jnp.asarray(PREV, jnp.int32)[my],
         my.astype(jnp.int32)]
    )
    origins_f = ring_arr[(r - 1 - hops) % N_DEV]
    origins_b = ring_arr[(r + 1 + hops) % N_DEV]

    def body(nbrs_ref, of_ref, ob_ref, x_ref, w_ref, out_ref,
             comm_f, comm_b, sf, rf, sb, rb, credit_f, credit_b):
        nxt = nbrs_ref[0]
        prv = nbrs_ref[1]
        mypos = nbrs_ref[2]

        barrier_sem = pltpu.get_barrier_semaphore()
        for nbr in (nxt, prv):
            pl.semaphore_signal(
                barrier_sem, inc=1,
                device_id=(nbr,), device_id_type=pl.DeviceIdType.MESH,
            )
        pl.semaphore_wait(barrier_sem, 2)

        def make(comm, ssem, rsem, h, dev):
            return pltpu.make_async_remote_copy(
                src_ref=comm.at[h % NSLOT],
                dst_ref=comm.at[(h + 1) % NSLOT],
                send_sem=ssem.at[h % NSLOT],
                recv_sem=rsem.at[(h + 1) % NSLOT],
                device_id=(dev,),
                device_id_type=pl.DeviceIdType.MESH,
            )

        def silu_gemm(chunk):
            y = jnp.dot(chunk, w_ref[...], preferred_element_type=jnp.float32)
            return y * (1.0 / (1.0 + jnp.exp(-y)))

        comm_f[0] = x_ref[0:m_half]
        comm_b[0] = x_ref[m_half:m_per]
        desc_f = make(comm_f, sf, rf, 0, nxt)
        desc_b = make(comm_b, sb, rb, 0, prv)
        desc_f.start()
        desc_b.start()
        out_ref[pl.ds(mypos * m_per, m_per), :] = silu_gemm(x_ref[...])

        for h in range(N_DEV - 1):
            desc_f.wait_recv()
            desc_b.wait_recv()
            desc_f.wait_send()
            desc_b.wait_send()
            if h < N_DEV - 2:
                if h + 1 >= NSLOT - 1:
                    pl.semaphore_wait(credit_f, 1)
                    pl.semaphore_wait(credit_b, 1)
                desc_f = make(comm_f, sf, rf, h + 1, nxt)
                desc_b = make(comm_b, sb, rb, h + 1, prv)
                desc_f.start()
                desc_b.start()
            if h + NSLOT - 1 < N_DEV - 1:
                pl.semaphore_signal(
                    credit_f, inc=1,
                    device_id=(prv,), device_id_type=pl.DeviceIdType.MESH,
                )
                pl.semaphore_signal(
                    credit_b, inc=1,
                    device_id=(nxt,), device_id_type=pl.DeviceIdType.MESH,
                )
            of = of_ref[h]
            ob = ob_ref[h]
            slot = (h + 1) % NSLOT
            out_ref[pl.ds(of * m_per, m_half), :] = silu_gemm(comm_f[slot])
            out_ref[pl.ds(ob * m_per + m_half, m_half), :] = silu_gemm(comm_b[slot])

    return pl.pallas_call(
        body,
        out_shape=jax.ShapeDtypeStruct((out_m, n_per), jnp.float32),
        in_specs=[
            pl.BlockSpec(memory_space=pltpu.SMEM),
            pl.BlockSpec(memory_space=pltpu.SMEM),
            pl.BlockSpec(memory_space=pltpu.SMEM),
            pl.BlockSpec(memory_space=pltpu.VMEM),
            pl.BlockSpec(memory_space=pltpu.VMEM),
        ],
        out_specs=pl.BlockSpec(memory_space=pltpu.VMEM),
        scratch_shapes=[
            pltpu.VMEM((NSLOT, m_half, k), jnp.bfloat16),
            pltpu.VMEM((NSLOT, m_half, k), jnp.bfloat16),
            pltpu.SemaphoreType.DMA((NSLOT,)),
            pltpu.SemaphoreType.DMA((NSLOT,)),
            pltpu.SemaphoreType.DMA((NSLOT,)),
            pltpu.SemaphoreType.DMA((NSLOT,)),
            pltpu.SemaphoreType.REGULAR,
            pltpu.SemaphoreType.REGULAR,
        ],
        compiler_params=pltpu.CompilerParams(
            collective_id=0, vmem_limit_bytes=100 * 1024 * 1024
        ),
    )(nbrs, origins_f, origins_b, x, w_mat)


# device time: 729329 ns/iter; 2.0147x vs baseline; 1.0219x over previous
import jax
import jax.numpy as jnp
from jax import lax
from jax.experimental import pallas as pl
from jax.experimental.pallas import tpu as pltpu

N_DEV = 16
NSLOT = 3

RING = [0, 1, 5, 9, 13, 14, 10, 6, 2, 3, 7, 11, 15, 12, 8, 4]
INV = [0] * N_DEV
for _j, _p in enumerate(RING):
    INV[_p] = _j
NEXT = [RING[(INV[p] + 1) % N_DEV] for p in range(N_DEV)]
PREV = [RING[(INV[p] - 1) % N_DEV] for p in range(N_DEV)]


def kernel(x, w_mat):
    w_mat = w_mat.astype(jnp.bfloat16)

    m_per, k = x.shape
    _, n_per = w_mat.shape
    m_half = m_per // 2
    out_m = N_DEV * m_per

    tbl = jnp.asarray([NEXT, PREV, INV, RING], jnp.int32)

    def body(tbl_ref, x_ref, w_ref, out_ref,
             comm_f, comm_b, sf, rf, sb, rb, credit_f, credit_b):
        my = lax.axis_index("i")
        nxt = tbl_ref[0, my]
        prv = tbl_ref[1, my]
        r = tbl_ref[2, my]

        comm_f[0] = x_ref[0:m_half].astype(jnp.bfloat16)
        comm_b[0] = x_ref[m_half:m_per].astype(jnp.bfloat16)

        barrier_sem = pltpu.get_barrier_semaphore()
        for nbr in (nxt, prv):
            pl.semaphore_signal(
                barrier_sem, inc=1,
                device_id=(nbr,), device_id_type=pl.DeviceIdType.MESH,
            )
        pl.semaphore_wait(barrier_sem, 2)

        def make(comm, ssem, rsem, h, dev):
            return pltpu.make_async_remote_copy(
                src_ref=comm.at[h % NSLOT],
                dst_ref=comm.at[(h + 1) % NSLOT],
                send_sem=ssem.at[h % NSLOT],
                recv_sem=rsem.at[(h + 1) % NSLOT],
                device_id=(dev,),
                device_id_type=pl.DeviceIdType.MESH,
            )

        def silu_gemm(chunk):
            y = jnp.dot(chunk, w_ref[...], preferred_element_type=jnp.float32)
            return y * (1.0 / (1.0 + jnp.exp(-y)))

        desc_f = make(comm_f, sf, rf, 0, nxt)
        desc_b = make(comm_b, sb, rb, 0, prv)
        desc_f.start()
        desc_b.start()
        out_ref[pl.ds(my * m_per, m_half), :] = silu_gemm(comm_f[0])
        out_ref[pl.ds(my * m_per + m_half, m_half), :] = silu_gemm(comm_b[0])

        for h in range(N_DEV - 1):
            desc_f.wait_recv()
            desc_b.wait_recv()
            desc_f.wait_send()
            desc_b.wait_send()
            if h < N_DEV - 2:
                if h + 1 >= NSLOT - 1:
                    pl.semaphore_wait(credit_f, 1)
                    pl.semaphore_wait(credit_b, 1)
                desc_f = make(comm_f, sf, rf, h + 1, nxt)
                desc_b = make(comm_b, sb, rb, h + 1, prv)
                desc_f.start()
                desc_b.start()
            if h + NSLOT - 1 < N_DEV - 1:
                pl.semaphore_signal(
                    credit_f, inc=1,
                    device_id=(prv,), device_id_type=pl.DeviceIdType.MESH,
                )
                pl.semaphore_signal(
                    credit_b, inc=1,
                    device_id=(nxt,), device_id_type=pl.DeviceIdType.MESH,
                )
            of = tbl_ref[3, (r + 2 * N_DEV - 1 - h) % N_DEV]
            ob = tbl_ref[3, (r + 1 + h) % N_DEV]
            slot = (h + 1) % NSLOT
            out_ref[pl.ds(of * m_per, m_half), :] = silu_gemm(comm_f[slot])
            out_ref[pl.ds(ob * m_per + m_half, m_half), :] = silu_gemm(comm_b[slot])

    return pl.pallas_call(
        body,
        out_shape=jax.ShapeDtypeStruct((out_m, n_per), jnp.float32),
        in_specs=[
            pl.BlockSpec(memory_space=pltpu.SMEM),
            pl.BlockSpec(memory_space=pltpu.VMEM),
            pl.BlockSpec(memory_space=pltpu.VMEM),
        ],
        out_specs=pl.BlockSpec(memory_space=pltpu.VMEM),
        scratch_shapes=[
            pltpu.VMEM((NSLOT, m_half, k), jnp.bfloat16),
            pltpu.VMEM((NSLOT, m_half, k), jnp.bfloat16),
            pltpu.SemaphoreType.DMA((NSLOT,)),
            pltpu.SemaphoreType.DMA((NSLOT,)),
            pltpu.SemaphoreType.DMA((NSLOT,)),
            pltpu.SemaphoreType.DMA((NSLOT,)),
            pltpu.SemaphoreType.REGULAR,
            pltpu.SemaphoreType.REGULAR,
        ],
        compiler_params=pltpu.CompilerParams(
            collective_id=0, vmem_limit_bytes=100 * 1024 * 1024
        ),
    )(tbl, x, w_mat)


# device time: 703754 ns/iter; 2.0879x vs baseline; 1.0363x over previous
import jax
import jax.numpy as jnp
from jax import lax
from jax.experimental import pallas as pl
from jax.experimental.pallas import tpu as pltpu

N_DEV = 16
NSLOT = 3
NSTREAM = 4

RING = [0, 1, 5, 9, 13, 14, 10, 6, 2, 3, 7, 11, 15, 12, 8, 4]
INV = [0] * N_DEV
for _j, _p in enumerate(RING):
    INV[_p] = _j
NEXT = [RING[(INV[p] + 1) % N_DEV] for p in range(N_DEV)]
PREV = [RING[(INV[p] - 1) % N_DEV] for p in range(N_DEV)]


def kernel(x, w_mat):
    w_mat = w_mat.astype(jnp.bfloat16)

    m_per, k = x.shape
    _, n_per = w_mat.shape
    m_q = m_per // 4
    out_m = N_DEV * m_per

    tbl = jnp.asarray([NEXT, PREV, INV, RING], jnp.int32)

    def body(tbl_ref, x_ref, w_ref, out_ref, comm, ssems, rsems, credits):
        my = lax.axis_index("i")
        nxt = tbl_ref[0, my]
        prv = tbl_ref[1, my]
        r = tbl_ref[2, my]

        dests = (nxt, nxt, prv, prv)
        creditees = (prv, prv, nxt, nxt)

        for q in range(NSTREAM):
            comm[q, 0] = x_ref[q * m_q:(q + 1) * m_q].astype(jnp.bfloat16)

        barrier_sem = pltpu.get_barrier_semaphore()
        for nbr in (nxt, prv):
            pl.semaphore_signal(
                barrier_sem, inc=1,
                device_id=(nbr,), device_id_type=pl.DeviceIdType.MESH,
            )
        pl.semaphore_wait(barrier_sem, 2)

        def make(q, h):
            return pltpu.make_async_remote_copy(
                src_ref=comm.at[q, h % NSLOT],
                dst_ref=comm.at[q, (h + 1) % NSLOT],
                send_sem=ssems.at[q, h % NSLOT],
                recv_sem=rsems.at[q, (h + 1) % NSLOT],
                device_id=(dests[q],),
                device_id_type=pl.DeviceIdType.MESH,
            )

        def silu_gemm(chunk):
            y = jnp.dot(chunk, w_ref[...], preferred_element_type=jnp.float32)
            return y * (1.0 / (1.0 + jnp.exp(-y)))

        def out_row(q, h):
            if q < 2:
                org = tbl_ref[3, (r + 2 * N_DEV - 1 - h) % N_DEV]
            else:
                org = tbl_ref[3, (r + 1 + h) % N_DEV]
            return org * m_per + q * m_q

        descs = [None] * NSTREAM
        for q in (0, 2, 1, 3):
            descs[q] = make(q, 0)
            descs[q].start()
        for q in range(NSTREAM):
            out_ref[pl.ds(my * m_per + q * m_q, m_q), :] = silu_gemm(comm[q, 0])

        for h in range(N_DEV - 1):
            for group in ((0, 2), (1, 3)):
                for q in group:
                    descs[q].wait_recv()
                    descs[q].wait_send()
                if h < N_DEV - 2:
                    for q in group:
                        if h + 1 >= NSLOT - 1:
                            pl.semaphore_wait(credits.at[q], 1)
                        descs[q] = make(q, h + 1)
                        descs[q].start()
                if h + NSLOT - 1 < N_DEV - 1:
                    for q in group:
                        pl.semaphore_signal(
                            credits.at[q], inc=1,
                            device_id=(creditees[q],),
                            device_id_type=pl.DeviceIdType.MESH,
                        )
                slot = (h + 1) % NSLOT
                for q in group:
                    out_ref[pl.ds(out_row(q, h), m_q), :] = silu_gemm(
                        comm[q, slot]
                    )

    return pl.pallas_call(
        body,
        out_shape=jax.ShapeDtypeStruct((out_m, n_per), jnp.float32),
        in_specs=[
            pl.BlockSpec(memory_space=pltpu.SMEM),
            pl.BlockSpec(memory_space=pltpu.VMEM),
            pl.BlockSpec(memory_space=pltpu.VMEM),
        ],
        out_specs=pl.BlockSpec(memory_space=pltpu.VMEM),
        scratch_shapes=[
            pltpu.VMEM((NSTREAM, NSLOT, m_q, k), jnp.bfloat16),
            pltpu.SemaphoreType.DMA((NSTREAM, NSLOT)),
            pltpu.SemaphoreType.DMA((NSTREAM, NSLOT)),
            pltpu.SemaphoreType.REGULAR((NSTREAM,)),
        ],
        compiler_params=pltpu.CompilerParams(
            collective_id=0, vmem_limit_bytes=100 * 1024 * 1024
        ),
    )(tbl, x, w_mat)


# device time: 703746 ns/iter; 2.0879x vs baseline; 1.0000x over previous
import jax
import jax.numpy as jnp
from jax import lax
from jax.experimental import pallas as pl
from jax.experimental.pallas import tpu as pltpu

N_DEV = 16
NSLOT = 3
NSTREAM = 4

RING = [0, 1, 5, 9, 13, 14, 10, 6, 2, 3, 7, 11, 15, 12, 8, 4]
INV = [0] * N_DEV
for _j, _p in enumerate(RING):
    INV[_p] = _j
NEXT = [RING[(INV[p] + 1) % N_DEV] for p in range(N_DEV)]
PREV = [RING[(INV[p] - 1) % N_DEV] for p in range(N_DEV)]


def kernel(x, w_mat):
    w_mat = w_mat.astype(jnp.bfloat16)

    m_per, k = x.shape
    _, n_per = w_mat.shape
    m_q = m_per // 4
    out_m = N_DEV * m_per

    tbl = jnp.asarray([NEXT, PREV, INV, RING], jnp.int32)

    def body(tbl_ref, x_ref, w_ref, out_ref, comm, ssems, rsems, credits):
        my = lax.axis_index("i")
        nxt = tbl_ref[0, my]
        prv = tbl_ref[1, my]
        r = tbl_ref[2, my]

        dests = (nxt, nxt, prv, prv)
        creditees = (prv, prv, nxt, nxt)

        for q in range(NSTREAM):
            comm[q, 0] = x_ref[q * m_q:(q + 1) * m_q].astype(jnp.bfloat16)

        barrier_sem = pltpu.get_barrier_semaphore()
        for nbr in (nxt, prv):
            pl.semaphore_signal(
                barrier_sem, inc=1,
                device_id=(nbr,), device_id_type=pl.DeviceIdType.MESH,
            )
        pl.semaphore_wait(barrier_sem, 2)

        def make(q, h):
            return pltpu.make_async_remote_copy(
                src_ref=comm.at[q, h % NSLOT],
                dst_ref=comm.at[q, (h + 1) % NSLOT],
                send_sem=ssems.at[q, h % NSLOT],
                recv_sem=rsems.at[q, (h + 1) % NSLOT],
                device_id=(dests[q],),
                device_id_type=pl.DeviceIdType.MESH,
            )

        def silu_gemm(chunk):
            y = jnp.dot(chunk, w_ref[...], preferred_element_type=jnp.float32)
            return y * (1.0 / (1.0 + jnp.exp(-y)))

        def out_row(q, h):
            if q < 2:
                org = tbl_ref[3, (r + 2 * N_DEV - 1 - h) % N_DEV]
            else:
                org = tbl_ref[3, (r + 1 + h) % N_DEV]
            return org * m_per + q * m_q

        for q in (0, 2, 1, 3):
            make(q, 0).start()
        for q in range(NSTREAM):
            out_ref[pl.ds(my * m_per + q * m_q, m_q), :] = silu_gemm(comm[q, 0])

        def hop(h, carry):
            for group in ((0, 2), (1, 3)):
                for q in group:
                    d = make(q, h)
                    d.wait_recv()
                    d.wait_send()

                @pl.when(h < N_DEV - 2)
                def _():
                    for q in group:
                        @pl.when(h + 1 >= NSLOT - 1)
                        def _():
                            pl.semaphore_wait(credits.at[q], 1)

                        make(q, h + 1).start()

                @pl.when(h + NSLOT - 1 < N_DEV - 1)
                def _():
                    for q in group:
                        pl.semaphore_signal(
                            credits.at[q], inc=1,
                            device_id=(creditees[q],),
                            device_id_type=pl.DeviceIdType.MESH,
                        )
                slot = (h + 1) % NSLOT
                for q in group:
                    out_ref[pl.ds(out_row(q, h), m_q), :] = silu_gemm(
                        comm[q, slot]
                    )
            return carry

        lax.fori_loop(0, N_DEV - 1, hop, 0)

    return pl.pallas_call(
        body,
        out_shape=jax.ShapeDtypeStruct((out_m, n_per), jnp.float32),
        in_specs=[
            pl.BlockSpec(memory_space=pltpu.SMEM),
            pl.BlockSpec(memory_space=pltpu.VMEM),
            pl.BlockSpec(memory_space=pltpu.VMEM),
        ],
        out_specs=pl.BlockSpec(memory_space=pltpu.VMEM),
        scratch_shapes=[
            pltpu.VMEM((NSTREAM, NSLOT, m_q, k), jnp.bfloat16),
            pltpu.SemaphoreType.DMA((NSTREAM, NSLOT)),
            pltpu.SemaphoreType.DMA((NSTREAM, NSLOT)),
            pltpu.SemaphoreType.REGULAR((NSTREAM,)),
        ],
        compiler_params=pltpu.CompilerParams(
            collective_id=0, vmem_limit_bytes=100 * 1024 * 1024
        ),
    )(tbl, x, w_mat)


# device time: 699315 ns/iter; 2.1011x vs baseline; 1.0063x over previous
import jax
import jax.numpy as jnp
from jax import lax
from jax.experimental import pallas as pl
from jax.experimental.pallas import tpu as pltpu

N_DEV = 16
NSLOT = 3
NSTREAM = 4

RING = [0, 1, 5, 9, 13, 14, 10, 6, 2, 3, 7, 11, 15, 12, 8, 4]
INV = [0] * N_DEV
for _j, _p in enumerate(RING):
    INV[_p] = _j
NEXT = [RING[(INV[p] + 1) % N_DEV] for p in range(N_DEV)]
PREV = [RING[(INV[p] - 1) % N_DEV] for p in range(N_DEV)]


def kernel(x, w_mat):
    m_per, k = x.shape
    _, n_per = w_mat.shape
    m_q = m_per // 4
    out_m = N_DEV * m_per

    tbl = jnp.asarray([NEXT, PREV, INV, RING], jnp.int32)

    def body(tbl_ref, x_ref, w_ref, out_ref, comm, wbf, ssems, rsems, credits):
        my = lax.axis_index("i")
        nxt = tbl_ref[0, my]
        prv = tbl_ref[1, my]
        r = tbl_ref[2, my]

        dests = (nxt, nxt, prv, prv)
        creditees = (prv, prv, nxt, nxt)

        for q in (0, 2):
            comm[q, 0] = x_ref[q * m_q:(q + 1) * m_q].astype(jnp.bfloat16)

        barrier_sem = pltpu.get_barrier_semaphore()
        for nbr in (nxt, prv):
            pl.semaphore_signal(
                barrier_sem, inc=1,
                device_id=(nbr,), device_id_type=pl.DeviceIdType.MESH,
            )
        pl.semaphore_wait(barrier_sem, 2)

        def make(q, h):
            return pltpu.make_async_remote_copy(
                src_ref=comm.at[q, h % NSLOT],
                dst_ref=comm.at[q, (h + 1) % NSLOT],
                send_sem=ssems.at[q, h % NSLOT],
                recv_sem=rsems.at[q, (h + 1) % NSLOT],
                device_id=(dests[q],),
                device_id_type=pl.DeviceIdType.MESH,
            )

        def silu_gemm(chunk):
            y = jnp.dot(chunk, wbf[...], preferred_element_type=jnp.float32)
            return y * (1.0 / (1.0 + jnp.exp(-y)))

        def out_row(q, h):
            if q < 2:
                org = tbl_ref[3, (r + 2 * N_DEV - 1 - h) % N_DEV]
            else:
                org = tbl_ref[3, (r + 1 + h) % N_DEV]
            return org * m_per + q * m_q

        for q in (0, 2):
            make(q, 0).start()
        for q in (1, 3):
            comm[q, 0] = x_ref[q * m_q:(q + 1) * m_q].astype(jnp.bfloat16)
            make(q, 0).start()
        wbf[...] = w_ref[...].astype(jnp.bfloat16)
        for q in range(NSTREAM):
            out_ref[pl.ds(my * m_per + q * m_q, m_q), :] = silu_gemm(comm[q, 0])

        def hop(h, carry):
            for group in ((0, 2), (1, 3)):
                for q in group:
                    d = make(q, h)
                    d.wait_recv()
                    d.wait_send()

                @pl.when(h < N_DEV - 2)
                def _():
                    for q in group:
                        @pl.when(h + 1 >= NSLOT - 1)
                        def _():
                            pl.semaphore_wait(credits.at[q], 1)

                        make(q, h + 1).start()

                @pl.when(h + NSLOT - 1 < N_DEV - 1)
                def _():
                    for q in group:
                        pl.semaphore_signal(
                            credits.at[q], inc=1,
                            device_id=(creditees[q],),
                            device_id_type=pl.DeviceIdType.MESH,
                        )
                slot = (h + 1) % NSLOT
                for q in group:
                    out_ref[pl.ds(out_row(q, h), m_q), :] = silu_gemm(
                        comm[q, slot]
                    )
            return carry

        lax.fori_loop(0, N_DEV - 1, hop, 0)

    return pl.pallas_call(
        body,
        out_shape=jax.ShapeDtypeStruct((out_m, n_per), jnp.float32),
        in_specs=[
            pl.BlockSpec(memory_space=pltpu.SMEM),
            pl.BlockSpec(memory_space=pltpu.VMEM),
            pl.BlockSpec(memory_space=pltpu.VMEM),
        ],
        out_specs=pl.BlockSpec(memory_space=pltpu.VMEM),
        scratch_shapes=[
            pltpu.VMEM((NSTREAM, NSLOT, m_q, k), jnp.bfloat16),
            pltpu.VMEM((k, n_per), jnp.bfloat16),
            pltpu.SemaphoreType.DMA((NSTREAM, NSLOT)),
            pltpu.SemaphoreType.DMA((NSTREAM, NSLOT)),
            pltpu.SemaphoreType.REGULAR((NSTREAM,)),
        ],
        compiler_params=pltpu.CompilerParams(
            collective_id=0, vmem_limit_bytes=100 * 1024 * 1024
        ),
    )(tbl, x, w_mat)
